# baseline (speedup 1.0000x reference)
import functools

import numpy as np
import jax
import jax.numpy as jnp
from jax.experimental import pallas as pl

_EPS = 1e-5
_CH = 8192
_FUSE_NP = 2048


def _offsets(Sp):
    out = []
    for a in (-1, 0, 1):
        for b in (-1, 0, 1):
            for c in (-1, 0, 1):
                out.append(a * Sp * Sp + b * Sp + c)
    return out


def _margins(S):
    Sp = S + 2
    M = Sp * Sp + Sp + 1
    M128 = -(-M // 128) * 128
    return M, M128


@functools.lru_cache(maxsize=None)
def _mask_np(S, Npr):
    Sp = S + 2
    m = np.zeros((Sp, Sp, Sp), np.float32)
    m[1:-1, 1:-1, 1:-1] = 1.0
    m = m.reshape(1, Sp ** 3)
    return np.pad(m, ((0, 0), (0, Npr - Sp ** 3)))


def _taps(x_ref, w_ref, base, n, ci, offs):
    acc = None
    for t, off in enumerate(offs):
        xs = x_ref[:, base + off: base + off + n]
        wt = w_ref[t]
        if ci == 1:
            contrib = wt.astype(jnp.float32) * xs.astype(jnp.float32)
        else:
            contrib = jax.lax.dot_general(
                wt, xs, (((1,), (0,)), ((), ())),
                preferred_element_type=jnp.float32)
        acc = contrib if acc is None else acc + contrib
    return acc


def _conv_small_kernel(x_ref, w_ref, g_ref, b_ref, mask_ref, o_ref, *,
                       ci, Np, M, Sp, N, do_bn, do_relu):
    acc = _taps(x_ref, w_ref, M, Np, ci, _offsets(Sp))
    mask = mask_ref[...]
    y = acc * mask
    if do_bn:
        m = jnp.sum(y, axis=1, keepdims=True) / N
        v = jnp.sum(y * y, axis=1, keepdims=True) / N - m * m
        y = ((y - m) * jax.lax.rsqrt(v + _EPS) * g_ref[...] + b_ref[...]) * mask
    else:
        y = y + b_ref[...]
    if do_relu:
        y = jnp.maximum(y, 0.0)
    o_ref[...] = y.astype(o_ref.dtype)


def _tree(parts):
    while len(parts) > 1:
        nxt = [parts[a] + parts[a + 1] for a in range(0, len(parts) - 1, 2)]
        if len(parts) % 2:
            nxt.append(parts[-1])
        parts = nxt
    return parts[0]


def _sumsq_tree(y, pieces=8):
    W = y.shape[1] // pieces
    p1, p2 = [], []
    for a in range(pieces):
        c = y[:, a * W:(a + 1) * W]
        p1.append(jnp.sum(c, axis=1, keepdims=True))
        p2.append(jnp.sum(c * c, axis=1, keepdims=True))
    return _tree(p1), _tree(p2)


def _conv_a_kernel(x_ref, w_ref, b_ref, mask_ref, y_ref, s_ref, *,
                   ci, M128, Sp, CH, fuse_bias):
    acc = _taps(x_ref, w_ref, M128, CH, ci, _offsets(Sp))
    if fuse_bias:
        y_ref[...] = acc + b_ref[...]
        s_ref[...] = jnp.zeros(s_ref.shape, s_ref.dtype)
        return
    y = acc * mask_ref[...]
    y_ref[...] = y
    s1, s2 = _sumsq_tree(y)
    s_ref[...] = jnp.concatenate([s1, s2], axis=1)[None]


def _bn_b_kernel(y_ref, s_ref, g_ref, b_ref, mask_ref, o_ref, *,
                 N, do_relu, nch):
    s = _tree([s_ref[k] for k in range(nch)])
    m = s[:, 0:1] / N
    v = s[:, 1:2] / N - m * m
    scale = jax.lax.rsqrt(v + _EPS) * g_ref[...]
    y = ((y_ref[...] - m) * scale + b_ref[...]) * mask_ref[...]
    if do_relu:
        y = jnp.maximum(y, 0.0)
    o_ref[...] = y.astype(o_ref.dtype)


def _conv_layer(x_pe, w, g, b, S, do_bn, do_relu):
    Sp = S + 2
    Np = Sp ** 3
    M, M128 = _margins(S)
    co, ci = int(w.shape[0]), int(w.shape[1])
    w27 = w.astype(jnp.bfloat16).reshape(co, ci, 27).transpose(2, 0, 1)
    g2 = g.reshape(-1, 1)
    b2 = b.reshape(-1, 1)
    odtype = jnp.bfloat16 if do_bn else jnp.float32

    if Np <= _FUSE_NP:
        mask = jnp.asarray(_mask_np(S, Np))
        kfn = functools.partial(_conv_small_kernel, ci=ci, Np=Np, M=M128,
                                Sp=Sp, N=S ** 3, do_bn=do_bn, do_relu=do_relu)
        return pl.pallas_call(
            kfn,
            out_shape=jax.ShapeDtypeStruct((co, Np), odtype),
        )(x_pe, w27, g2, b2, mask)

    nch = -(-Np // _CH)
    Npr = nch * _CH
    CHW = _CH + 2 * M128
    mask = jnp.asarray(_mask_np(S, Npr))

    a_fn = functools.partial(_conv_a_kernel, ci=ci, M128=M128, Sp=Sp,
                             CH=_CH, fuse_bias=not do_bn)
    y, s = pl.pallas_call(
        a_fn,
        grid=(nch,),
        in_specs=[
            pl.BlockSpec((pl.Element(ci), pl.Element(CHW)),
                         lambda i: (0, i * _CH)),
            pl.BlockSpec((27, co, ci), lambda i: (0, 0, 0)),
            pl.BlockSpec((co, 1), lambda i: (0, 0)),
            pl.BlockSpec((1, _CH), lambda i: (0, i)),
        ],
        out_specs=[
            pl.BlockSpec((co, _CH), lambda i: (0, i)),
            pl.BlockSpec((1, co, 2), lambda i: (i, 0, 0)),
        ],
        out_shape=[
            jax.ShapeDtypeStruct((co, Npr), jnp.float32),
            jax.ShapeDtypeStruct((nch, co, 2), jnp.float32),
        ],
    )(x_pe, w27, b2, mask)
    if not do_bn:
        return y[:, :Np]

    b_fn = functools.partial(_bn_b_kernel, N=S ** 3, do_relu=do_relu,
                             nch=nch)
    out = pl.pallas_call(
        b_fn,
        grid=(nch,),
        in_specs=[
            pl.BlockSpec((co, _CH), lambda i: (0, i)),
            pl.BlockSpec((nch, co, 2), lambda i: (0, 0, 0)),
            pl.BlockSpec((co, 1), lambda i: (0, 0)),
            pl.BlockSpec((co, 1), lambda i: (0, 0)),
            pl.BlockSpec((1, _CH), lambda i: (0, i)),
        ],
        out_specs=pl.BlockSpec((co, _CH), lambda i: (0, i)),
        out_shape=jax.ShapeDtypeStruct((co, Npr), jnp.bfloat16),
    )(y, s, g2, b2, mask)
    return out[:, :Np]


def _pool_kernel(x_ref, o_ref):
    o_ref[...] = jnp.max(x_ref[...], axis=0)


def _pool(v4):
    c, S = int(v4.shape[0]), int(v4.shape[1])
    S2 = S // 2
    n2 = S2 ** 3
    slabs = [v4[:, a::2, b::2, cc::2].reshape(c, n2)
             for a in (0, 1) for b in (0, 1) for cc in (0, 1)]
    xs = jnp.stack(slabs, axis=0)
    ch = min(n2, _CH)
    nch = n2 // ch
    out = pl.pallas_call(
        _pool_kernel,
        grid=(nch,),
        in_specs=[pl.BlockSpec((8, c, ch), lambda i: (0, 0, i))],
        out_specs=pl.BlockSpec((c, ch), lambda i: (0, i)),
        out_shape=jax.ShapeDtypeStruct((c, n2), v4.dtype),
    )(xs)
    return out.reshape(c, S2, S2, S2)


def _up_a_kernel(x_ref, w_ref, y_ref, s_ref, *, pieces):
    p1, p2 = [], []
    for j in range(8):
        yj = jax.lax.dot_general(
            w_ref[j], x_ref[...], (((1,), (0,)), ((), ())),
            preferred_element_type=jnp.float32)
        y_ref[j] = yj
        s1, s2 = _sumsq_tree(yj, pieces)
        p1.append(s1)
        p2.append(s2)
    s_ref[...] = jnp.concatenate([_tree(p1), _tree(p2)], axis=1)[None]


def _up_b_kernel(y_ref, s_ref, g_ref, b_ref, o_ref, *, n, nch):
    s = _tree([s_ref[k] for k in range(nch)])
    m = (s[:, 0:1] / (8 * n)).reshape(1, -1, 1)
    v = (s[:, 1:2] / (8 * n)).reshape(1, -1, 1) - m * m
    scale = jax.lax.rsqrt(v + _EPS) * g_ref[...]
    o_ref[...] = jnp.maximum((y_ref[...] - m) * scale + b_ref[...],
                             0.0).astype(o_ref.dtype)


def _upconv_layer(x4, wu, g, b):
    ci, S = int(x4.shape[0]), int(x4.shape[1])
    co = int(wu.shape[0])
    n = S ** 3
    w8 = wu.astype(jnp.bfloat16).reshape(co, ci, 8)[:, :, ::-1].transpose(2, 0, 1)
    xf = x4.reshape(ci, n)
    ch = min(n, _CH)
    nch = n // ch
    y, s = pl.pallas_call(
        functools.partial(_up_a_kernel, pieces=min(8, max(1, ch // 256))),
        grid=(nch,),
        in_specs=[
            pl.BlockSpec((ci, ch), lambda i: (0, i)),
            pl.BlockSpec((8, co, ci), lambda i: (0, 0, 0)),
        ],
        out_specs=[
            pl.BlockSpec((8, co, ch), lambda i: (0, 0, i)),
            pl.BlockSpec((1, co, 2), lambda i: (i, 0, 0)),
        ],
        out_shape=[
            jax.ShapeDtypeStruct((8, co, n), jnp.float32),
            jax.ShapeDtypeStruct((nch, co, 2), jnp.float32),
        ],
    )(xf, w8)
    out = pl.pallas_call(
        functools.partial(_up_b_kernel, n=n, nch=nch),
        grid=(nch,),
        in_specs=[
            pl.BlockSpec((8, co, ch), lambda i: (0, 0, i)),
            pl.BlockSpec((nch, co, 2), lambda i: (0, 0, 0)),
            pl.BlockSpec((1, co, 1), lambda i: (0, 0, 0)),
            pl.BlockSpec((1, co, 1), lambda i: (0, 0, 0)),
        ],
        out_specs=pl.BlockSpec((8, co, ch), lambda i: (0, 0, i)),
        out_shape=jax.ShapeDtypeStruct((8, co, n), jnp.bfloat16),
    )(y, s, g.reshape(1, -1, 1), b.reshape(1, -1, 1))
    yg = out.reshape(2, 2, 2, co, S, S, S)
    return yg.transpose(3, 4, 0, 5, 1, 6, 2).reshape(co, 2 * S, 2 * S, 2 * S)


def _npr(S):
    Np = (S + 2) ** 3
    return Np if Np <= _FUSE_NP else -(-Np // _CH) * _CH


def _to_pe_from4(x4):
    c, S = int(x4.shape[0]), int(x4.shape[1])
    Sp = S + 2
    Np = Sp ** 3
    _, M128 = _margins(S)
    xp = jnp.pad(x4, ((0, 0), (1, 1), (1, 1), (1, 1))).reshape(c, Np)
    return jnp.pad(xp, ((0, 0), (M128, _npr(S) - Np + M128)))


def _to_pe_from_flat(y, S):
    Sp = S + 2
    Np = Sp ** 3
    _, M128 = _margins(S)
    return jnp.pad(y, ((0, 0), (M128, _npr(S) - Np + M128)))


def _crop(y, S):
    c = int(y.shape[0])
    Sp = S + 2
    return y.reshape(c, Sp, Sp, Sp)[:, 1:-1, 1:-1, 1:-1]


def kernel(x, params):
    p = params
    h = x[0].astype(jnp.bfloat16)

    skips = {}
    S = 64
    for i in (1, 2, 3, 4):
        x_pe = _to_pe_from4(h)
        y = _conv_layer(x_pe, p['d%d_w1' % i], p['d%d_g1' % i],
                        p['d%d_b1' % i], S, do_bn=True, do_relu=True)
        y = _conv_layer(_to_pe_from_flat(y, S), p['d%d_w2' % i],
                        p['d%d_g2' % i], p['d%d_b2' % i], S,
                        do_bn=True, do_relu=False)
        v4 = _crop(y, S)
        skips[i] = v4
        h = _pool(v4)
        S //= 2

    y = _conv_layer(_to_pe_from4(h), p['bn_w1'], p['bn_g1'], p['bn_b1'],
                    S, do_bn=True, do_relu=False)
    y = _conv_layer(_to_pe_from_flat(y, S), p['bn_w2'], p['bn_g2'],
                    p['bn_b2'], S, do_bn=True, do_relu=False)
    h = _crop(y, S)

    for i in (4, 3, 2, 1):
        up = _upconv_layer(h, p['u%d_wu' % i], p['u%d_gu' % i],
                           p['u%d_bu' % i])
        S *= 2
        cat = jnp.concatenate([up, skips[i]], axis=0)
        y = _conv_layer(_to_pe_from4(cat), p['u%d_w1' % i], p['u%d_g1' % i],
                        p['u%d_b1' % i], S, do_bn=True, do_relu=True)
        y = _conv_layer(_to_pe_from_flat(y, S), p['u%d_w2' % i],
                        p['u%d_g2' % i], p['u%d_b2' % i], S,
                        do_bn=True, do_relu=True)
        h = _crop(y, S)

    y = _conv_layer(_to_pe_from4(h), p['pf_w'], p['pf_g'], p['pf_b'],
                    S, do_bn=True, do_relu=False)
    y = _conv_layer(_to_pe_from_flat(y, S), p['fin_w'],
                    jnp.ones((20,), jnp.float32), p['fin_b'], S,
                    do_bn=False, do_relu=False)
    out = _crop(y, S)
    return out[None].astype(jnp.float32)


# speedup vs baseline: 1.0148x; 1.0148x over previous
import functools

import numpy as np
import jax
import jax.numpy as jnp
from jax.experimental import pallas as pl

_EPS = 1e-5
_CH = 16384
_FUSE_NP = 2048


def _offsets(Sp):
    out = []
    for a in (-1, 0, 1):
        for b in (-1, 0, 1):
            for c in (-1, 0, 1):
                out.append(a * Sp * Sp + b * Sp + c)
    return out


def _margins(S):
    Sp = S + 2
    M = Sp * Sp + Sp + 1
    M128 = -(-M // 128) * 128
    return M, M128


@functools.lru_cache(maxsize=None)
def _mask_np(S, Npr):
    Sp = S + 2
    m = np.zeros((Sp, Sp, Sp), np.float32)
    m[1:-1, 1:-1, 1:-1] = 1.0
    m = m.reshape(1, Sp ** 3)
    return np.pad(m, ((0, 0), (0, Npr - Sp ** 3)))


def _taps(x_ref, w_ref, base, n, ci, offs):
    acc = None
    for t, off in enumerate(offs):
        xs = x_ref[:, base + off: base + off + n]
        wt = w_ref[t]
        if ci == 1:
            contrib = wt.astype(jnp.float32) * xs.astype(jnp.float32)
        else:
            contrib = jax.lax.dot_general(
                wt, xs, (((1,), (0,)), ((), ())),
                preferred_element_type=jnp.float32)
        acc = contrib if acc is None else acc + contrib
    return acc


def _conv_small_kernel(x_ref, w_ref, g_ref, b_ref, mask_ref, o_ref, *,
                       ci, Np, M, Sp, N, do_bn, do_relu):
    acc = _taps(x_ref, w_ref, M, Np, ci, _offsets(Sp))
    mask = mask_ref[...]
    y = acc * mask
    if do_bn:
        m = jnp.sum(y, axis=1, keepdims=True) / N
        v = jnp.sum(y * y, axis=1, keepdims=True) / N - m * m
        y = ((y - m) * jax.lax.rsqrt(v + _EPS) * g_ref[...] + b_ref[...]) * mask
    else:
        y = y + b_ref[...]
    if do_relu:
        y = jnp.maximum(y, 0.0)
    o_ref[...] = y.astype(o_ref.dtype)


def _tree(parts):
    while len(parts) > 1:
        nxt = [parts[a] + parts[a + 1] for a in range(0, len(parts) - 1, 2)]
        if len(parts) % 2:
            nxt.append(parts[-1])
        parts = nxt
    return parts[0]


def _sumsq_tree(y, pieces=8):
    W = y.shape[1] // pieces
    p1, p2 = [], []
    for a in range(pieces):
        c = y[:, a * W:(a + 1) * W]
        p1.append(jnp.sum(c, axis=1, keepdims=True))
        p2.append(jnp.sum(c * c, axis=1, keepdims=True))
    return _tree(p1), _tree(p2)


def _conv_a_kernel(x_ref, w_ref, b_ref, mask_ref, y_ref, s_ref, *,
                   ci, M128, Sp, CH, fuse_bias):
    acc = _taps(x_ref, w_ref, M128, CH, ci, _offsets(Sp))
    if fuse_bias:
        y_ref[...] = acc + b_ref[...]
        s_ref[...] = jnp.zeros(s_ref.shape, s_ref.dtype)
        return
    y = acc * mask_ref[...]
    y_ref[...] = y
    s1, s2 = _sumsq_tree(y)
    s_ref[...] = jnp.concatenate([s1, s2], axis=1)[None]


def _bn_b_kernel(y_ref, s_ref, g_ref, b_ref, mask_ref, o_ref, *,
                 N, do_relu, nch):
    s = _tree([s_ref[k] for k in range(nch)])
    m = s[:, 0:1] / N
    v = s[:, 1:2] / N - m * m
    scale = jax.lax.rsqrt(v + _EPS) * g_ref[...]
    y = ((y_ref[...] - m) * scale + b_ref[...]) * mask_ref[...]
    if do_relu:
        y = jnp.maximum(y, 0.0)
    o_ref[...] = y.astype(o_ref.dtype)


def _conv_layer(x_pe, w, g, b, S, do_bn, do_relu):
    Sp = S + 2
    Np = Sp ** 3
    M, M128 = _margins(S)
    co, ci = int(w.shape[0]), int(w.shape[1])
    w27 = w.astype(jnp.bfloat16).reshape(co, ci, 27).transpose(2, 0, 1)
    g2 = g.reshape(-1, 1)
    b2 = b.reshape(-1, 1)
    odtype = jnp.bfloat16 if do_bn else jnp.float32

    if Np <= _FUSE_NP:
        mask = jnp.asarray(_mask_np(S, Np))
        kfn = functools.partial(_conv_small_kernel, ci=ci, Np=Np, M=M128,
                                Sp=Sp, N=S ** 3, do_bn=do_bn, do_relu=do_relu)
        return pl.pallas_call(
            kfn,
            out_shape=jax.ShapeDtypeStruct((co, Np), odtype),
        )(x_pe, w27, g2, b2, mask)

    nch = -(-Np // _CH)
    Npr = nch * _CH
    CHW = _CH + 2 * M128
    mask = jnp.asarray(_mask_np(S, Npr))

    a_fn = functools.partial(_conv_a_kernel, ci=ci, M128=M128, Sp=Sp,
                             CH=_CH, fuse_bias=not do_bn)
    y, s = pl.pallas_call(
        a_fn,
        grid=(nch,),
        in_specs=[
            pl.BlockSpec((pl.Element(ci), pl.Element(CHW)),
                         lambda i: (0, i * _CH)),
            pl.BlockSpec((27, co, ci), lambda i: (0, 0, 0)),
            pl.BlockSpec((co, 1), lambda i: (0, 0)),
            pl.BlockSpec((1, _CH), lambda i: (0, i)),
        ],
        out_specs=[
            pl.BlockSpec((co, _CH), lambda i: (0, i)),
            pl.BlockSpec((1, co, 2), lambda i: (i, 0, 0)),
        ],
        out_shape=[
            jax.ShapeDtypeStruct((co, Npr), jnp.float32),
            jax.ShapeDtypeStruct((nch, co, 2), jnp.float32),
        ],
    )(x_pe, w27, b2, mask)
    if not do_bn:
        return y[:, :Np]

    b_fn = functools.partial(_bn_b_kernel, N=S ** 3, do_relu=do_relu,
                             nch=nch)
    out = pl.pallas_call(
        b_fn,
        grid=(nch,),
        in_specs=[
            pl.BlockSpec((co, _CH), lambda i: (0, i)),
            pl.BlockSpec((nch, co, 2), lambda i: (0, 0, 0)),
            pl.BlockSpec((co, 1), lambda i: (0, 0)),
            pl.BlockSpec((co, 1), lambda i: (0, 0)),
            pl.BlockSpec((1, _CH), lambda i: (0, i)),
        ],
        out_specs=pl.BlockSpec((co, _CH), lambda i: (0, i)),
        out_shape=jax.ShapeDtypeStruct((co, Npr), jnp.bfloat16),
    )(y, s, g2, b2, mask)
    return out[:, :Np]


def _pool_kernel(x_ref, o_ref):
    o_ref[...] = jnp.max(x_ref[...], axis=0)


def _pool(v4):
    c, S = int(v4.shape[0]), int(v4.shape[1])
    S2 = S // 2
    n2 = S2 ** 3
    slabs = [v4[:, a::2, b::2, cc::2].reshape(c, n2)
             for a in (0, 1) for b in (0, 1) for cc in (0, 1)]
    xs = jnp.stack(slabs, axis=0)
    ch = min(n2, _CH)
    nch = n2 // ch
    out = pl.pallas_call(
        _pool_kernel,
        grid=(nch,),
        in_specs=[pl.BlockSpec((8, c, ch), lambda i: (0, 0, i))],
        out_specs=pl.BlockSpec((c, ch), lambda i: (0, i)),
        out_shape=jax.ShapeDtypeStruct((c, n2), v4.dtype),
    )(xs)
    return out.reshape(c, S2, S2, S2)


def _up_a_kernel(x_ref, w_ref, y_ref, s_ref, *, pieces):
    p1, p2 = [], []
    for j in range(8):
        yj = jax.lax.dot_general(
            w_ref[j], x_ref[...], (((1,), (0,)), ((), ())),
            preferred_element_type=jnp.float32)
        y_ref[j] = yj
        s1, s2 = _sumsq_tree(yj, pieces)
        p1.append(s1)
        p2.append(s2)
    s_ref[...] = jnp.concatenate([_tree(p1), _tree(p2)], axis=1)[None]


def _up_b_kernel(y_ref, s_ref, g_ref, b_ref, o_ref, *, n, nch):
    s = _tree([s_ref[k] for k in range(nch)])
    m = (s[:, 0:1] / (8 * n)).reshape(1, -1, 1)
    v = (s[:, 1:2] / (8 * n)).reshape(1, -1, 1) - m * m
    scale = jax.lax.rsqrt(v + _EPS) * g_ref[...]
    o_ref[...] = jnp.maximum((y_ref[...] - m) * scale + b_ref[...],
                             0.0).astype(o_ref.dtype)


def _upconv_layer(x4, wu, g, b):
    ci, S = int(x4.shape[0]), int(x4.shape[1])
    co = int(wu.shape[0])
    n = S ** 3
    w8 = wu.astype(jnp.bfloat16).reshape(co, ci, 8)[:, :, ::-1].transpose(2, 0, 1)
    xf = x4.reshape(ci, n)
    ch = min(n, _CH)
    nch = n // ch
    y, s = pl.pallas_call(
        functools.partial(_up_a_kernel, pieces=min(8, max(1, ch // 256))),
        grid=(nch,),
        in_specs=[
            pl.BlockSpec((ci, ch), lambda i: (0, i)),
            pl.BlockSpec((8, co, ci), lambda i: (0, 0, 0)),
        ],
        out_specs=[
            pl.BlockSpec((8, co, ch), lambda i: (0, 0, i)),
            pl.BlockSpec((1, co, 2), lambda i: (i, 0, 0)),
        ],
        out_shape=[
            jax.ShapeDtypeStruct((8, co, n), jnp.float32),
            jax.ShapeDtypeStruct((nch, co, 2), jnp.float32),
        ],
    )(xf, w8)
    out = pl.pallas_call(
        functools.partial(_up_b_kernel, n=n, nch=nch),
        grid=(nch,),
        in_specs=[
            pl.BlockSpec((8, co, ch), lambda i: (0, 0, i)),
            pl.BlockSpec((nch, co, 2), lambda i: (0, 0, 0)),
            pl.BlockSpec((1, co, 1), lambda i: (0, 0, 0)),
            pl.BlockSpec((1, co, 1), lambda i: (0, 0, 0)),
        ],
        out_specs=pl.BlockSpec((8, co, ch), lambda i: (0, 0, i)),
        out_shape=jax.ShapeDtypeStruct((8, co, n), jnp.bfloat16),
    )(y, s, g.reshape(1, -1, 1), b.reshape(1, -1, 1))
    yg = out.reshape(2, 2, 2, co, S, S, S)
    return yg.transpose(3, 4, 0, 5, 1, 6, 2).reshape(co, 2 * S, 2 * S, 2 * S)


def _npr(S):
    Np = (S + 2) ** 3
    return Np if Np <= _FUSE_NP else -(-Np // _CH) * _CH


def _to_pe_from4(x4):
    c, S = int(x4.shape[0]), int(x4.shape[1])
    Sp = S + 2
    Np = Sp ** 3
    _, M128 = _margins(S)
    xp = jnp.pad(x4, ((0, 0), (1, 1), (1, 1), (1, 1))).reshape(c, Np)
    return jnp.pad(xp, ((0, 0), (M128, _npr(S) - Np + M128)))


def _to_pe_from_flat(y, S):
    Sp = S + 2
    Np = Sp ** 3
    _, M128 = _margins(S)
    return jnp.pad(y, ((0, 0), (M128, _npr(S) - Np + M128)))


def _crop(y, S):
    c = int(y.shape[0])
    Sp = S + 2
    return y.reshape(c, Sp, Sp, Sp)[:, 1:-1, 1:-1, 1:-1]


def kernel(x, params):
    p = params
    h = x[0].astype(jnp.bfloat16)

    skips = {}
    S = 64
    for i in (1, 2, 3, 4):
        x_pe = _to_pe_from4(h)
        y = _conv_layer(x_pe, p['d%d_w1' % i], p['d%d_g1' % i],
                        p['d%d_b1' % i], S, do_bn=True, do_relu=True)
        y = _conv_layer(_to_pe_from_flat(y, S), p['d%d_w2' % i],
                        p['d%d_g2' % i], p['d%d_b2' % i], S,
                        do_bn=True, do_relu=False)
        v4 = _crop(y, S)
        skips[i] = v4
        h = _pool(v4)
        S //= 2

    y = _conv_layer(_to_pe_from4(h), p['bn_w1'], p['bn_g1'], p['bn_b1'],
                    S, do_bn=True, do_relu=False)
    y = _conv_layer(_to_pe_from_flat(y, S), p['bn_w2'], p['bn_g2'],
                    p['bn_b2'], S, do_bn=True, do_relu=False)
    h = _crop(y, S)

    for i in (4, 3, 2, 1):
        up = _upconv_layer(h, p['u%d_wu' % i], p['u%d_gu' % i],
                           p['u%d_bu' % i])
        S *= 2
        cat = jnp.concatenate([up, skips[i]], axis=0)
        y = _conv_layer(_to_pe_from4(cat), p['u%d_w1' % i], p['u%d_g1' % i],
                        p['u%d_b1' % i], S, do_bn=True, do_relu=True)
        y = _conv_layer(_to_pe_from_flat(y, S), p['u%d_w2' % i],
                        p['u%d_g2' % i], p['u%d_b2' % i], S,
                        do_bn=True, do_relu=True)
        h = _crop(y, S)

    y = _conv_layer(_to_pe_from4(h), p['pf_w'], p['pf_g'], p['pf_b'],
                    S, do_bn=True, do_relu=False)
    y = _conv_layer(_to_pe_from_flat(y, S), p['fin_w'],
                    jnp.ones((20,), jnp.float32), p['fin_b'], S,
                    do_bn=False, do_relu=False)
    out = _crop(y, S)
    return out[None].astype(jnp.float32)
